# baseline (speedup 1.0000x reference)
import functools

import jax
import jax.numpy as jnp
import numpy as np
from jax import lax
from jax.experimental import pallas as pl
from jax.experimental.pallas import tpu as pltpu
from jax.experimental.pallas import tpu_sc as plsc

NUM_CORES = 2
NUM_SUBCORES = 16
NW = NUM_CORES * NUM_SUBCORES
LANES = 16

F = 26
V = 100000
D = 64
B = 4096
R = B * F
B_W = B // NW
ROWS_W = B_W * F
CHUNK = 128
NCHUNK = ROWS_W // CHUNK

_OFFPAIR = np.concatenate([
    np.arange(16, dtype=np.int64) * V,
    np.arange(10, 26, dtype=np.int64) * V,
]).astype(np.int32)

_mesh = plsc.VectorSubcoreMesh(core_axis_name="c", subcore_axis_name="s")


@functools.partial(
    pl.kernel,
    mesh=_mesh,
    compiler_params=pltpu.CompilerParams(use_tc_tiling_on_sc=False),
    out_type=jax.ShapeDtypeStruct((R, D), jnp.float32),
    scratch_types=[
        pltpu.VMEM((B_W, 128), jnp.int32),
        pltpu.VMEM((32,), jnp.int32),
        pltpu.VMEM((ROWS_W,), jnp.int32),
        pltpu.VMEM((CHUNK, D), jnp.float32),
        pltpu.VMEM((CHUNK, D), jnp.float32),
        pltpu.SemaphoreType.DMA,
        pltpu.SemaphoreType.DMA,
    ],
)
def _sc_gather(x_hbm, offpair_hbm, tab_hbm, out_hbm,
               xblk_v, offpair_v, xflat, buf0, buf1, gsem0, gsem1):
    wid = lax.axis_index("s") * NUM_CORES + lax.axis_index("c")
    out_base = wid * ROWS_W

    pltpu.sync_copy(x_hbm.at[pl.ds(wid * B_W, B_W), :], xblk_v)
    pltpu.sync_copy(offpair_hbm, offpair_v)

    offa = offpair_v[pl.ds(0, LANES)]
    offb = offpair_v[pl.ds(LANES, LANES)]

    def repack_body(r, _):
        p = r * F
        xflat[pl.ds(p, LANES)] = xblk_v[r, pl.ds(0, LANES)] + offa
        xflat[pl.ds(p + 10, LANES)] = xblk_v[r, pl.ds(10, LANES)] + offb
        return 0

    lax.fori_loop(0, B_W, repack_body, 0)

    def gather_start(c, buf, sem):
        pltpu.make_async_copy(
            tab_hbm.at[xflat.at[pl.ds(c * CHUNK, CHUNK)]], buf, sem).start()

    def gather_wait(c, buf, sem):
        pltpu.make_async_copy(
            tab_hbm.at[xflat.at[pl.ds(c * CHUNK, CHUNK)]], buf, sem).wait()

    def scatter(c, buf):
        pltpu.sync_copy(buf, out_hbm.at[pl.ds(out_base + c * CHUNK, CHUNK)])

    gather_start(0, buf0, gsem0)
    gather_start(1, buf1, gsem1)

    def loop_body(i, _):
        for b, (buf, sem) in enumerate(((buf0, gsem0), (buf1, gsem1))):
            c = 2 * i + b
            gather_wait(c, buf, sem)
            scatter(c, buf)
            gather_start(c + 2, buf, sem)
        return 0

    lax.fori_loop(0, (NCHUNK - 2) // 2, loop_body, 0)

    for b, (buf, sem) in enumerate(((buf0, gsem0), (buf1, gsem1))):
        c = NCHUNK - 2 + b
        gather_wait(c, buf, sem)
        scatter(c, buf)


def kernel(X, tables):
    x = jnp.pad(jnp.asarray(X, jnp.int32), ((0, 0), (0, 128 - F)))
    tab = tables.reshape(F * V, D)
    out_flat = _sc_gather(x, jnp.asarray(_OFFPAIR), tab)
    return out_flat.reshape(B, F, D)


# speedup vs baseline: 2.5603x; 2.5603x over previous
import functools

import jax
import jax.numpy as jnp
import numpy as np
from jax import lax
from jax.experimental import pallas as pl
from jax.experimental.pallas import tpu as pltpu
from jax.experimental.pallas import tpu_sc as plsc

NUM_CORES = 2
NUM_SUBCORES = 16
NW = NUM_CORES * NUM_SUBCORES
LANES = 16

F = 26
V = 100000
D = 64
B = 4096
R = B * F
B_W = B // NW
ROWS_W = B_W * F
CHUNK = 128
NCHUNK = ROWS_W // CHUNK

_OFFPAIR = np.concatenate([
    np.arange(16, dtype=np.int64) * V,
    np.arange(10, 26, dtype=np.int64) * V,
]).astype(np.int32)

_mesh = plsc.VectorSubcoreMesh(core_axis_name="c", subcore_axis_name="s")


@functools.partial(
    pl.kernel,
    mesh=_mesh,
    compiler_params=pltpu.CompilerParams(use_tc_tiling_on_sc=True),
    out_type=jax.ShapeDtypeStruct((R, D), jnp.float32),
    scratch_types=[
        pltpu.VMEM((B_W, 128), jnp.int32),
        pltpu.VMEM((32,), jnp.int32),
        pltpu.VMEM((ROWS_W,), jnp.int32),
        pltpu.VMEM((CHUNK, D), jnp.float32),
        pltpu.VMEM((CHUNK, D), jnp.float32),
        pltpu.SemaphoreType.DMA,
        pltpu.SemaphoreType.DMA,
    ],
)
def _sc_gather(x_hbm, offpair_hbm, tab_hbm, out_hbm,
               xblk_v, offpair_v, xflat, buf0, buf1, gsem0, gsem1):
    wid = lax.axis_index("s") * NUM_CORES + lax.axis_index("c")
    out_base = wid * ROWS_W

    pltpu.sync_copy(x_hbm.at[pl.ds(wid * B_W, B_W), :], xblk_v)
    pltpu.sync_copy(offpair_hbm, offpair_v)

    offa = offpair_v[pl.ds(0, LANES)]
    offb = offpair_v[pl.ds(LANES, LANES)]

    def repack_body(r, _):
        p = r * F
        xflat[pl.ds(p, LANES)] = xblk_v[r, pl.ds(0, LANES)] + offa
        xflat[pl.ds(p + 10, LANES)] = xblk_v[r, pl.ds(10, LANES)] + offb
        return 0

    lax.fori_loop(0, B_W, repack_body, 0)

    def gather_start(c, buf, sem):
        def group_body(g, _):
            vec = xflat[pl.ds(c * CHUNK + g * LANES, LANES)]
            for l in range(LANES):
                r = vec[l]
                pltpu.make_async_copy(
                    tab_hbm.at[pl.ds(r, 1), :],
                    buf.at[pl.ds(g * LANES + l, 1), :], sem
                ).start()
            return 0
        lax.fori_loop(0, CHUNK // LANES, group_body, 0)

    def gather_wait(buf, sem):
        pltpu.make_async_copy(
            tab_hbm.at[pl.ds(0, CHUNK), :], buf, sem).wait()

    def scatter(c, buf):
        pltpu.sync_copy(buf, out_hbm.at[pl.ds(out_base + c * CHUNK, CHUNK)])

    gather_start(0, buf0, gsem0)
    gather_start(1, buf1, gsem1)

    def loop_body(i, _):
        for b, (buf, sem) in enumerate(((buf0, gsem0), (buf1, gsem1))):
            c = 2 * i + b
            gather_wait(buf, sem)
            scatter(c, buf)
            gather_start(c + 2, buf, sem)
        return 0

    lax.fori_loop(0, (NCHUNK - 2) // 2, loop_body, 0)

    for b, (buf, sem) in enumerate(((buf0, gsem0), (buf1, gsem1))):
        c = NCHUNK - 2 + b
        gather_wait(buf, sem)
        scatter(c, buf)


def kernel(X, tables):
    x = jnp.pad(jnp.asarray(X, jnp.int32), ((0, 0), (0, 128 - F)))
    tab = tables.reshape(F * V, D)
    out_flat = _sc_gather(x, jnp.asarray(_OFFPAIR), tab)
    return out_flat.reshape(B, F, D)


# speedup vs baseline: 2.7192x; 1.0621x over previous
import functools

import jax
import jax.numpy as jnp
import numpy as np
from jax import lax
from jax.experimental import pallas as pl
from jax.experimental.pallas import tpu as pltpu
from jax.experimental.pallas import tpu_sc as plsc

NUM_CORES = 2
NUM_SUBCORES = 16
NW = NUM_CORES * NUM_SUBCORES
LANES = 16

F = 26
V = 100000
D = 64
B = 4096
B_W = B // NW
ROWS_W = B_W * F
CB = 8
CROWS = CB * F
NCHUNK = B_W // CB

_OFFPAIR = np.concatenate([
    np.arange(16, dtype=np.int64) * V,
    np.arange(10, 26, dtype=np.int64) * V,
]).astype(np.int32)

_mesh = plsc.VectorSubcoreMesh(core_axis_name="c", subcore_axis_name="s")


@functools.partial(
    pl.kernel,
    mesh=_mesh,
    compiler_params=pltpu.CompilerParams(use_tc_tiling_on_sc=True),
    out_type=jax.ShapeDtypeStruct((B, F, D), jnp.float32),
    scratch_types=[
        pltpu.VMEM((B_W, 128), jnp.int32),
        pltpu.VMEM((32,), jnp.int32),
        pltpu.VMEM((ROWS_W,), jnp.int32),
        pltpu.VMEM((CB, F, D), jnp.float32),
        pltpu.VMEM((CB, F, D), jnp.float32),
        pltpu.SemaphoreType.DMA,
        pltpu.SemaphoreType.DMA,
    ],
)
def _sc_gather(x_hbm, offpair_hbm, tab_hbm, out_hbm,
               xblk_v, offpair_v, xflat, buf0, buf1, gsem0, gsem1):
    wid = lax.axis_index("s") * NUM_CORES + lax.axis_index("c")
    b_base = wid * B_W

    pltpu.sync_copy(x_hbm.at[pl.ds(b_base, B_W), :], xblk_v)
    pltpu.sync_copy(offpair_hbm, offpair_v)

    offa = offpair_v[pl.ds(0, LANES)]
    offb = offpair_v[pl.ds(LANES, LANES)]

    def repack_body(r, _):
        p = r * F
        xflat[pl.ds(p, LANES)] = xblk_v[r, pl.ds(0, LANES)] + offa
        xflat[pl.ds(p + 10, LANES)] = xblk_v[r, pl.ds(10, LANES)] + offb
        return 0

    lax.fori_loop(0, B_W, repack_body, 0)

    def gather_start(c, buf, sem):
        def group_body(g, _):
            q0 = g * LANES
            vec = xflat[pl.ds(c * CROWS + q0, LANES)]
            for l in range(LANES):
                r = vec[l]
                q = q0 + l
                bq = q // F
                fq = q - bq * F
                pltpu.make_async_copy(
                    tab_hbm.at[r], buf.at[bq, fq], sem).start()
            return 0
        lax.fori_loop(0, CROWS // LANES, group_body, 0)

    def gather_wait(buf, sem):
        pltpu.make_async_copy(out_hbm.at[pl.ds(0, CB), :, :], buf, sem).wait()

    def scatter(c, buf):
        pltpu.sync_copy(buf, out_hbm.at[pl.ds(b_base + c * CB, CB), :, :])

    gather_start(0, buf0, gsem0)
    gather_start(1, buf1, gsem1)

    def loop_body(i, _):
        for b, (buf, sem) in enumerate(((buf0, gsem0), (buf1, gsem1))):
            c = 2 * i + b
            gather_wait(buf, sem)
            scatter(c, buf)
            gather_start(c + 2, buf, sem)
        return 0

    lax.fori_loop(0, (NCHUNK - 2) // 2, loop_body, 0)

    for b, (buf, sem) in enumerate(((buf0, gsem0), (buf1, gsem1))):
        c = NCHUNK - 2 + b
        gather_wait(buf, sem)
        scatter(c, buf)


def kernel(X, tables):
    x = jnp.pad(jnp.asarray(X, jnp.int32), ((0, 0), (0, 128 - F)))
    tab = tables.reshape(F * V, D)
    return _sc_gather(x, jnp.asarray(_OFFPAIR), tab)
